# baseline (speedup 1.0000x reference)
import functools

import jax
import jax.numpy as jnp
from jax import lax
from jax.experimental import pallas as pl
from jax.experimental.pallas import tpu as pltpu
from jax.experimental.pallas import tpu_sc as plsc

_TILE = 256
_K = 10
_KP = 16


def _tc_body(q_ref, k_ref, ow_ref, oi_ref):
    q = q_ref[0]
    k = k_ref[0, 0]

    qn = q / jnp.sqrt(jnp.sum(q * q, axis=1, keepdims=True))
    kn = k / jnp.sqrt(jnp.sum(k * k, axis=1, keepdims=True))
    p = jax.lax.dot_general(
        qn, kn, (((1,), (1,)), ((), ())),
        preferred_element_type=jnp.float32,
    )

    n = p.shape[1]
    iota_f = jnp.broadcast_to(
        jax.lax.broadcasted_iota(jnp.int32, (1, p.shape[1]), 1)
        .astype(jnp.float32), p.shape)
    big = float(n)
    v0 = jnp.max(p, axis=1)
    ws, idxs = [], []
    denom = jnp.zeros_like(v0)
    for _ in range(_K):
        vmax = jnp.max(p, axis=1)
        w = jnp.exp(vmax - v0)
        idx = jnp.min(jnp.where(p == vmax[:, None], iota_f, big), axis=1)
        sel = iota_f == idx[:, None]
        p = jnp.where(sel, -jnp.inf, p)
        ws.append(w)
        idxs.append(idx)
        denom = denom + w

    inv = 1.0 / denom
    w_mat = jnp.stack([w * inv for w in ws]
                      + [jnp.zeros_like(v0)] * (_KP - _K), axis=0)
    i_mat = jnp.stack(idxs + [jnp.zeros_like(v0)] * (_KP - _K),
                      axis=0).astype(jnp.int32)
    ow_ref[0, 0] = w_mat
    oi_ref[0, 0] = i_mat


def _topk_weights(source_feat, target_feat):
    b, n, f = source_feat.shape
    q = jnp.concatenate([target_feat, source_feat], axis=1)
    keys = jnp.stack([source_feat, target_feat], axis=1)

    nt = n // _TILE
    return pl.pallas_call(
        _tc_body,
        grid=(b, 2, nt),
        compiler_params=pltpu.CompilerParams(
            dimension_semantics=("parallel", "parallel", "parallel")),
        in_specs=[
            pl.BlockSpec((1, _TILE, f), lambda bi, s, i: (bi, s * nt + i, 0)),
            pl.BlockSpec((1, 1, n, f), lambda bi, s, i: (bi, s, 0, 0)),
        ],
        out_specs=[
            pl.BlockSpec((1, 1, _KP, _TILE), lambda bi, s, i: (bi, s, 0, i)),
            pl.BlockSpec((1, 1, _KP, _TILE), lambda bi, s, i: (bi, s, 0, i)),
        ],
        out_shape=[
            jax.ShapeDtypeStruct((b, 2, _KP, n), jnp.float32),
            jax.ShapeDtypeStruct((b, 2, _KP, n), jnp.int32),
        ],
    )(q, keys)


def _sc_recon(n_groups, n, r_total, rows_per_tec, w_hbm, i_hbm, pos_hbm,
              out_hbm, w_v, i_v, px_v, py_v, pz_v, ox_v, oy_v, oz_v):
    c = lax.axis_index("c")
    s = lax.axis_index("s")
    wid = s * 2 + c
    tecs_per_group = 32 // n_groups
    g = wid // tecs_per_group
    t = wid % tecs_per_group
    r0 = t * rows_per_tec

    pltpu.sync_copy(w_hbm.at[g, :, pl.ds(r0, rows_per_tec)], w_v)
    pltpu.sync_copy(i_hbm.at[g, :, pl.ds(r0, rows_per_tec)], i_v)
    pltpu.sync_copy(pos_hbm.at[pl.ds((g * 3 + 0) * n, n)], px_v)
    pltpu.sync_copy(pos_hbm.at[pl.ds((g * 3 + 1) * n, n)], py_v)
    pltpu.sync_copy(pos_hbm.at[pl.ds((g * 3 + 2) * n, n)], pz_v)

    def body(i, carry):
        base = i * 16
        accx = jnp.zeros((16,), jnp.float32)
        accy = jnp.zeros((16,), jnp.float32)
        accz = jnp.zeros((16,), jnp.float32)
        for kk in range(_K):
            wv = w_v[kk, pl.ds(base, 16)]
            iv = i_v[kk, pl.ds(base, 16)]
            accx = accx + wv * plsc.load_gather(px_v, [iv])
            accy = accy + wv * plsc.load_gather(py_v, [iv])
            accz = accz + wv * plsc.load_gather(pz_v, [iv])
        ox_v[pl.ds(base, 16)] = accx
        oy_v[pl.ds(base, 16)] = accy
        oz_v[pl.ds(base, 16)] = accz
        return carry

    lax.fori_loop(0, rows_per_tec // 16, body, 0)

    pltpu.sync_copy(ox_v, out_hbm.at[pl.ds((g * 3 + 0) * r_total + r0, rows_per_tec)])
    pltpu.sync_copy(oy_v, out_hbm.at[pl.ds((g * 3 + 1) * r_total + r0, rows_per_tec)])
    pltpu.sync_copy(oz_v, out_hbm.at[pl.ds((g * 3 + 2) * r_total + r0, rows_per_tec)])


def _recon(w, idx, pos_flat):
    g_, _, r_ = w.shape
    n = pos_flat.shape[0] // (g_ * 3)
    rows_per_tec = (g_ * r_) // 32
    mesh = plsc.VectorSubcoreMesh(core_axis_name="c", subcore_axis_name="s")
    fn = functools.partial(
        pl.kernel,
        mesh=mesh,
        compiler_params=pltpu.CompilerParams(needs_layout_passes=False),
        out_type=jax.ShapeDtypeStruct((g_ * 3 * r_,), jnp.float32),
        scratch_types=[
            pltpu.VMEM((_KP, rows_per_tec), jnp.float32),
            pltpu.VMEM((_KP, rows_per_tec), jnp.int32),
            pltpu.VMEM((n,), jnp.float32),
            pltpu.VMEM((n,), jnp.float32),
            pltpu.VMEM((n,), jnp.float32),
            pltpu.VMEM((rows_per_tec,), jnp.float32),
            pltpu.VMEM((rows_per_tec,), jnp.float32),
            pltpu.VMEM((rows_per_tec,), jnp.float32),
        ],
    )(functools.partial(_sc_recon, g_, n, r_, rows_per_tec))
    return fn(w, idx, pos_flat)


@jax.jit
def kernel(source, target, source_feat, target_feat):
    b, n, f = source_feat.shape
    w, idx = _topk_weights(source_feat, target_feat)
    w = w.reshape(2 * b, _KP, n)
    idx = idx.reshape(2 * b, _KP, n)
    pos = jnp.stack([source, target], axis=1)
    pos_flat = jnp.swapaxes(pos, 2, 3).reshape(2 * b * 3 * n)
    out = _recon(w, idx, pos_flat)
    out = out.reshape(2 * b, 3, n)
    return jnp.swapaxes(out, 1, 2).reshape(b, 2 * n, 3)


# speedup vs baseline: 1.0502x; 1.0502x over previous
import functools

import jax
import jax.numpy as jnp
from jax import lax
from jax.experimental import pallas as pl
from jax.experimental.pallas import tpu as pltpu
from jax.experimental.pallas import tpu_sc as plsc

_TILE = 256
_K = 10
_KP = 16


def _norm_body(x_ref, o_ref):
    x = x_ref[0, 0]
    o_ref[0, 0] = x / jnp.sqrt(jnp.sum(x * x, axis=1, keepdims=True))


def _tc_body(q_ref, k_ref, ow_ref, oi_ref):
    qn = q_ref[0]
    kn = k_ref[0, 0]

    p = jax.lax.dot_general(
        qn, kn, (((1,), (1,)), ((), ())),
        preferred_element_type=jnp.float32,
    )

    n = p.shape[1]
    iota_f = jnp.broadcast_to(
        jax.lax.broadcasted_iota(jnp.int32, (1, p.shape[1]), 1)
        .astype(jnp.float32), p.shape)
    big = float(n)
    v0 = jnp.max(p, axis=1)
    ws, idxs = [], []
    denom = jnp.zeros_like(v0)
    for _ in range(_K):
        vmax = jnp.max(p, axis=1)
        w = jnp.exp(vmax - v0)
        idx = jnp.min(jnp.where(p == vmax[:, None], iota_f, big), axis=1)
        sel = iota_f == idx[:, None]
        p = jnp.where(sel, -jnp.inf, p)
        ws.append(w)
        idxs.append(idx)
        denom = denom + w

    inv = 1.0 / denom
    w_mat = jnp.stack([w * inv for w in ws]
                      + [jnp.zeros_like(v0)] * (_KP - _K), axis=0)
    i_mat = jnp.stack(idxs + [jnp.zeros_like(v0)] * (_KP - _K),
                      axis=0).astype(jnp.int32)
    ow_ref[0, 0] = w_mat
    oi_ref[0, 0] = i_mat


def _topk_weights(source_feat, target_feat):
    b, n, f = source_feat.shape
    feats = jnp.stack([source_feat, target_feat], axis=1)
    keys = pl.pallas_call(
        _norm_body,
        grid=(b, 2),
        in_specs=[pl.BlockSpec((1, 1, n, f), lambda bi, s: (bi, s, 0, 0))],
        out_specs=pl.BlockSpec((1, 1, n, f), lambda bi, s: (bi, s, 0, 0)),
        out_shape=jax.ShapeDtypeStruct((b, 2, n, f), jnp.float32),
    )(feats)
    q = jnp.concatenate([keys[:, 1], keys[:, 0]], axis=1)

    nt = n // _TILE
    return pl.pallas_call(
        _tc_body,
        grid=(b, 2, nt),
        compiler_params=pltpu.CompilerParams(
            dimension_semantics=("parallel", "parallel", "parallel")),
        in_specs=[
            pl.BlockSpec((1, _TILE, f), lambda bi, s, i: (bi, s * nt + i, 0)),
            pl.BlockSpec((1, 1, n, f), lambda bi, s, i: (bi, s, 0, 0)),
        ],
        out_specs=[
            pl.BlockSpec((1, 1, _KP, _TILE), lambda bi, s, i: (bi, s, 0, i)),
            pl.BlockSpec((1, 1, _KP, _TILE), lambda bi, s, i: (bi, s, 0, i)),
        ],
        out_shape=[
            jax.ShapeDtypeStruct((b, 2, _KP, n), jnp.float32),
            jax.ShapeDtypeStruct((b, 2, _KP, n), jnp.int32),
        ],
    )(q, keys)


def _sc_recon(n_groups, n, r_total, rows_per_tec, w_hbm, i_hbm, pos_hbm,
              out_hbm, w_v, i_v, px_v, py_v, pz_v, ox_v, oy_v, oz_v):
    c = lax.axis_index("c")
    s = lax.axis_index("s")
    wid = s * 2 + c
    tecs_per_group = 32 // n_groups
    g = wid // tecs_per_group
    t = wid % tecs_per_group
    r0 = t * rows_per_tec

    pltpu.sync_copy(w_hbm.at[g, :, pl.ds(r0, rows_per_tec)], w_v)
    pltpu.sync_copy(i_hbm.at[g, :, pl.ds(r0, rows_per_tec)], i_v)
    pltpu.sync_copy(pos_hbm.at[pl.ds((g * 3 + 0) * n, n)], px_v)
    pltpu.sync_copy(pos_hbm.at[pl.ds((g * 3 + 1) * n, n)], py_v)
    pltpu.sync_copy(pos_hbm.at[pl.ds((g * 3 + 2) * n, n)], pz_v)

    def body(i, carry):
        base = i * 16
        accx = jnp.zeros((16,), jnp.float32)
        accy = jnp.zeros((16,), jnp.float32)
        accz = jnp.zeros((16,), jnp.float32)
        for kk in range(_K):
            wv = w_v[kk, pl.ds(base, 16)]
            iv = i_v[kk, pl.ds(base, 16)]
            accx = accx + wv * plsc.load_gather(px_v, [iv])
            accy = accy + wv * plsc.load_gather(py_v, [iv])
            accz = accz + wv * plsc.load_gather(pz_v, [iv])
        ox_v[pl.ds(base, 16)] = accx
        oy_v[pl.ds(base, 16)] = accy
        oz_v[pl.ds(base, 16)] = accz
        return carry

    lax.fori_loop(0, rows_per_tec // 16, body, 0)

    pltpu.sync_copy(ox_v, out_hbm.at[pl.ds((g * 3 + 0) * r_total + r0, rows_per_tec)])
    pltpu.sync_copy(oy_v, out_hbm.at[pl.ds((g * 3 + 1) * r_total + r0, rows_per_tec)])
    pltpu.sync_copy(oz_v, out_hbm.at[pl.ds((g * 3 + 2) * r_total + r0, rows_per_tec)])


def _recon(w, idx, pos_flat):
    g_, _, r_ = w.shape
    n = pos_flat.shape[0] // (g_ * 3)
    rows_per_tec = (g_ * r_) // 32
    mesh = plsc.VectorSubcoreMesh(core_axis_name="c", subcore_axis_name="s")
    fn = functools.partial(
        pl.kernel,
        mesh=mesh,
        compiler_params=pltpu.CompilerParams(needs_layout_passes=False),
        out_type=jax.ShapeDtypeStruct((g_ * 3 * r_,), jnp.float32),
        scratch_types=[
            pltpu.VMEM((_KP, rows_per_tec), jnp.float32),
            pltpu.VMEM((_KP, rows_per_tec), jnp.int32),
            pltpu.VMEM((n,), jnp.float32),
            pltpu.VMEM((n,), jnp.float32),
            pltpu.VMEM((n,), jnp.float32),
            pltpu.VMEM((rows_per_tec,), jnp.float32),
            pltpu.VMEM((rows_per_tec,), jnp.float32),
            pltpu.VMEM((rows_per_tec,), jnp.float32),
        ],
    )(functools.partial(_sc_recon, g_, n, r_, rows_per_tec))
    return fn(w, idx, pos_flat)


@jax.jit
def kernel(source, target, source_feat, target_feat):
    b, n, f = source_feat.shape
    w, idx = _topk_weights(source_feat, target_feat)
    w = w.reshape(2 * b, _KP, n)
    idx = idx.reshape(2 * b, _KP, n)
    pos = jnp.stack([source, target], axis=1)
    pos_flat = jnp.swapaxes(pos, 2, 3).reshape(2 * b * 3 * n)
    out = _recon(w, idx, pos_flat)
    out = out.reshape(2 * b, 3, n)
    return jnp.swapaxes(out, 1, 2).reshape(b, 2 * n, 3)
